# Initial kernel scaffold; baseline (speedup 1.0000x reference)
import jax
import jax.numpy as jnp
from jax.experimental import pallas as pl


def kernel(x, edge_index, edge_weight, W):
    raise NotImplementedError("write your pallas kernel here")


# speedup vs baseline: 5.0971x; 5.0971x over previous
import functools

import jax
import jax.numpy as jnp
from jax import lax
from jax.experimental import pallas as pl
from jax.experimental.pallas import tpu as pltpu
from jax.experimental.pallas import tpu_sc as plsc

N = 10000
NPAD = 10240
D = 128
E = 320000
C = 128
NCHUNK = E // C
NW = 32
FULL_T = NCHUNK // NW
REM = NCHUNK - FULL_T * NW
ROWS_PER_TILE = NPAD // 16
ZROWS = 128


def _sc_body(x_hbm, src_hbm, dst_hbm, w_hbm, part_hbm,
             idx_v, dst_v, w_v, rows_v, zbuf_v, acc_sh, sem):
  c = lax.axis_index("c")
  s = lax.axis_index("s")
  wid = s * 2 + c

  def zrow(i, carry):
    for j in range(8):
      zbuf_v[i, pl.ds(j * 16, 16)] = jnp.zeros((16,), jnp.float32)
    return carry

  lax.fori_loop(0, ZROWS, zrow, 0)
  for r in range(ROWS_PER_TILE // ZROWS):
    pltpu.sync_copy(zbuf_v, acc_sh.at[pl.ds(s * ROWS_PER_TILE + r * ZROWS, ZROWS)])
  plsc.subcore_barrier()

  def do_chunk(k):
    off = k * C
    pltpu.sync_copy(src_hbm.at[pl.ds(off, C)], idx_v)
    pltpu.sync_copy(dst_hbm.at[pl.ds(off, C)], dst_v)
    pltpu.sync_copy(w_hbm.at[pl.ds(off, C)], w_v)
    pltpu.async_copy(x_hbm.at[idx_v], rows_v, sem).wait()

    def scale16(i, carry):
      wv = w_v[pl.ds(i * 16, 16)]
      for r in range(16):
        wb = jnp.broadcast_to(wv[r], (16,))
        row = i * 16 + r
        for j in range(8):
          sl = pl.ds(j * 16, 16)
          rows_v[row, sl] = rows_v[row, sl] * wb
      return carry

    lax.fori_loop(0, C // 16, scale16, 0)
    pltpu.sync_copy(rows_v, acc_sh.at[dst_v], add=True)

  def round_body(t, carry):
    do_chunk(wid + NW * t)
    return carry

  lax.fori_loop(0, FULL_T, round_body, 0)

  @pl.when(wid < REM)
  def _():
    do_chunk(FULL_T * NW + wid)

  plsc.subcore_barrier()
  base = s * ROWS_PER_TILE
  pltpu.sync_copy(acc_sh.at[pl.ds(base, ROWS_PER_TILE)],
                  part_hbm.at[c, pl.ds(base, ROWS_PER_TILE)])


_sc_agg = functools.partial(
    pl.kernel,
    out_type=jax.ShapeDtypeStruct((2, NPAD, D), jnp.float32),
    mesh=plsc.VectorSubcoreMesh(core_axis_name="c", subcore_axis_name="s"),
    scratch_types=[
        pltpu.VMEM((C,), jnp.int32),
        pltpu.VMEM((C,), jnp.int32),
        pltpu.VMEM((C,), jnp.float32),
        pltpu.VMEM((C, D), jnp.float32),
        pltpu.VMEM((ZROWS, D), jnp.float32),
        pltpu.VMEM_SHARED((NPAD, D), jnp.float32),
        pltpu.SemaphoreType.DMA,
    ],
)(_sc_body)


BM = 1000


def _mm_body(p_ref, w_ref, o_ref):
  acc = p_ref[0] + p_ref[1]
  o_ref[...] = jnp.dot(acc, w_ref[...], preferred_element_type=jnp.float32)


def _combine_matmul(part, W):
  return pl.pallas_call(
      _mm_body,
      grid=(N // BM,),
      in_specs=[
          pl.BlockSpec((2, BM, D), lambda i: (0, i, 0)),
          pl.BlockSpec((D, D), lambda i: (0, 0)),
      ],
      out_specs=pl.BlockSpec((BM, D), lambda i: (i, 0)),
      out_shape=jax.ShapeDtypeStruct((N, D), jnp.float32),
  )(part, W)


@jax.jit
def kernel(x, edge_index, edge_weight, W):
  dst = edge_index[0].astype(jnp.int32)
  src = edge_index[1].astype(jnp.int32)
  part = _sc_agg(x, src, dst, edge_weight)
  return _combine_matmul(part, W)


# speedup vs baseline: 8.3620x; 1.6405x over previous
import functools

import jax
import jax.numpy as jnp
from jax import lax
from jax.experimental import pallas as pl
from jax.experimental.pallas import tpu as pltpu
from jax.experimental.pallas import tpu_sc as plsc

N = 10000
NPAD = 10240
D = 128
E = 320000
C = 128
NCHUNK = E // C
NW = 32
FULL_T = NCHUNK // NW
REM = NCHUNK - FULL_T * NW
ROWS_PER_TILE = NPAD // 16
TSEG = 26
NSEG = FULL_T // TSEG
TBUF = TSEG * C


def _sc_body(x_hbm, src_hbm, dst_hbm, w_hbm, part_hbm,
             src_all, dst_all, w_all, rows0, rows1, dstc0, dstc1, acc_sh,
             gsem0, gsem1, ssem0, ssem1):
  c = lax.axis_index("c")
  s = lax.axis_index("s")
  wid = s * 2 + c
  base = (wid * FULL_T + jnp.minimum(wid, REM)) * C

  def zrow(i, carry):
    for j in range(8):
      rows0[i, pl.ds(j * 16, 16)] = jnp.zeros((16,), jnp.float32)
    return carry

  lax.fori_loop(0, C, zrow, 0)
  zd = []
  for r in range(ROWS_PER_TILE // C):
    zd.append(pltpu.async_copy(
        rows0, acc_sh.at[pl.ds(s * ROWS_PER_TILE + r * C, C)], ssem1))
  for dsc in zd:
    dsc.wait()

  plsc.subcore_barrier()

  def scale_chunk(buf, k):
    def scale16(i, carry):
      wv = w_all[pl.ds(k * C + i * 16, 16)]
      for r in range(16):
        wb = jnp.broadcast_to(wv[r], (16,))
        row = i * 16 + r
        for j in range(8):
          sl = pl.ds(j * 16, 16)
          buf[row, sl] = buf[row, sl] * wb
      return carry

    lax.fori_loop(0, C // 16, scale16, 0)

  def stage_dst(dstc, k):
    for j in range(8):
      dstc[pl.ds(j * 16, 16)] = dst_all[pl.ds(k * C + j * 16, 16)]

  def seg_body(g, carry):
    sbase = base + g * TBUF
    pltpu.sync_copy(src_hbm.at[pl.ds(sbase, TBUF)], src_all)
    pltpu.sync_copy(dst_hbm.at[pl.ds(sbase, TBUF)], dst_all)
    pltpu.sync_copy(w_hbm.at[pl.ds(sbase, TBUF)], w_all)
    pltpu.async_copy(x_hbm.at[src_all.at[pl.ds(0, C)]], rows0, gsem0)
    pltpu.async_copy(x_hbm.at[src_all.at[pl.ds(C, C)]], rows1, gsem1)

    def iter_body(t2, carry2):
      k0 = 2 * t2
      k1 = k0 + 1
      pltpu.make_async_copy(x_hbm.at[src_all.at[pl.ds(k0 * C, C)]], rows0,
                            gsem0).wait()
      scale_chunk(rows0, k0)
      stage_dst(dstc0, k0)
      pltpu.async_copy(rows0, acc_sh.at[dstc0], ssem0, add=True)
      pltpu.make_async_copy(x_hbm.at[src_all.at[pl.ds(k1 * C, C)]], rows1,
                            gsem1).wait()
      scale_chunk(rows1, k1)
      stage_dst(dstc1, k1)
      pltpu.async_copy(rows1, acc_sh.at[dstc1], ssem1, add=True)
      pltpu.make_async_copy(rows0, acc_sh.at[dstc0], ssem0).wait()
      pltpu.make_async_copy(rows1, acc_sh.at[dstc1], ssem1).wait()

      @pl.when(t2 < TSEG // 2 - 1)
      def _():
        pltpu.async_copy(x_hbm.at[src_all.at[pl.ds((k0 + 2) * C, C)]], rows0,
                         gsem0)
        pltpu.async_copy(x_hbm.at[src_all.at[pl.ds((k1 + 2) * C, C)]], rows1,
                         gsem1)

      return carry2

    lax.fori_loop(0, TSEG // 2, iter_body, 0)
    return carry

  lax.fori_loop(0, NSEG, seg_body, 0)

  @pl.when(wid < REM)
  def _():
    roff = base + FULL_T * C
    pltpu.sync_copy(src_hbm.at[pl.ds(roff, C)], src_all.at[pl.ds(0, C)])
    pltpu.sync_copy(dst_hbm.at[pl.ds(roff, C)], dst_all.at[pl.ds(0, C)])
    pltpu.sync_copy(w_hbm.at[pl.ds(roff, C)], w_all.at[pl.ds(0, C)])
    pltpu.async_copy(x_hbm.at[src_all.at[pl.ds(0, C)]], rows0, gsem0).wait()
    scale_chunk(rows0, 0)
    stage_dst(dstc0, 0)
    pltpu.sync_copy(rows0, acc_sh.at[dstc0], add=True)

  plsc.subcore_barrier()
  rbase = s * ROWS_PER_TILE
  pltpu.sync_copy(acc_sh.at[pl.ds(rbase, ROWS_PER_TILE)],
                  part_hbm.at[c, pl.ds(rbase, ROWS_PER_TILE)])


_sc_agg = functools.partial(
    pl.kernel,
    out_type=jax.ShapeDtypeStruct((2, NPAD, D), jnp.float32),
    mesh=plsc.VectorSubcoreMesh(core_axis_name="c", subcore_axis_name="s"),
    scratch_types=[
        pltpu.VMEM((TBUF,), jnp.int32),
        pltpu.VMEM((TBUF,), jnp.int32),
        pltpu.VMEM((TBUF,), jnp.float32),
        pltpu.VMEM((C, D), jnp.float32),
        pltpu.VMEM((C, D), jnp.float32),
        pltpu.VMEM((C,), jnp.int32),
        pltpu.VMEM((C,), jnp.int32),
        pltpu.VMEM_SHARED((NPAD, D), jnp.float32),
        pltpu.SemaphoreType.DMA,
        pltpu.SemaphoreType.DMA,
        pltpu.SemaphoreType.DMA,
        pltpu.SemaphoreType.DMA,
    ],
)(_sc_body)


BM = 1000


def _mm_body(p_ref, w_ref, o_ref):
  acc = p_ref[0] + p_ref[1]
  o_ref[...] = jnp.dot(acc, w_ref[...], preferred_element_type=jnp.float32)


def _combine_matmul(part, W):
  return pl.pallas_call(
      _mm_body,
      grid=(N // BM,),
      in_specs=[
          pl.BlockSpec((2, BM, D), lambda i: (0, i, 0)),
          pl.BlockSpec((D, D), lambda i: (0, 0)),
      ],
      out_specs=pl.BlockSpec((BM, D), lambda i: (i, 0)),
      out_shape=jax.ShapeDtypeStruct((N, D), jnp.float32),
  )(part, W)


@jax.jit
def kernel(x, edge_index, edge_weight, W):
  dst = edge_index[0].astype(jnp.int32)
  src = edge_index[1].astype(jnp.int32)
  part = _sc_agg(x, src, dst, edge_weight)
  return _combine_matmul(part, W)


# speedup vs baseline: 10.5669x; 1.2637x over previous
import functools

import jax
import jax.numpy as jnp
from jax import lax
from jax.experimental import pallas as pl
from jax.experimental.pallas import tpu as pltpu
from jax.experimental.pallas import tpu_sc as plsc

N = 10000
NPAD = 10240
D = 128
E = 320000
C = 64
NCHUNK = E // C
NW = 32
FULL_T = 156
REM = NCHUNK - FULL_T * NW
ROWS_PER_TILE = NPAD // 16
TSEG = 52
NSEG = FULL_T // TSEG
TBUF = TSEG * C
NSLOT = 4


def _sc_body(x_hbm, src_hbm, dst_hbm, w_hbm, part_hbm,
             src_all, dst_all, w_all,
             rows0, rows1, rows2, rows3, dstc0, dstc1, dstc2, dstc3, acc_sh,
             gsem0, gsem1, gsem2, gsem3, ssem0, ssem1, ssem2, ssem3):
  rows = [rows0, rows1, rows2, rows3]
  dstc = [dstc0, dstc1, dstc2, dstc3]
  gsem = [gsem0, gsem1, gsem2, gsem3]
  ssem = [ssem0, ssem1, ssem2, ssem3]
  c = lax.axis_index("c")
  s = lax.axis_index("s")
  wid = s * 2 + c
  base = (wid * FULL_T + jnp.minimum(wid, REM)) * C

  def zrow(i, carry):
    for j in range(8):
      rows0[i, pl.ds(j * 16, 16)] = jnp.zeros((16,), jnp.float32)
    return carry

  lax.fori_loop(0, C, zrow, 0)
  zd = []
  for r in range(ROWS_PER_TILE // C):
    zd.append(pltpu.async_copy(
        rows0, acc_sh.at[pl.ds(s * ROWS_PER_TILE + r * C, C)], ssem0))
  for dsc in zd:
    dsc.wait()

  plsc.subcore_barrier()

  def scale_chunk(buf, k):
    def scale16(i, carry):
      wv = w_all[pl.ds(k * C + i * 16, 16)]
      for r in range(16):
        wb = jnp.broadcast_to(wv[r], (16,))
        row = i * 16 + r
        for j in range(8):
          sl = pl.ds(j * 16, 16)
          buf[row, sl] = buf[row, sl] * wb
      return carry

    lax.fori_loop(0, C // 16, scale16, 0)

  def stage_dst(b, k):
    for j in range(C // 16):
      dstc[b][pl.ds(j * 16, 16)] = dst_all[pl.ds(k * C + j * 16, 16)]

  def start_gather(k, b):
    pltpu.async_copy(x_hbm.at[src_all.at[pl.ds(k * C, C)]], rows[b], gsem[b])

  def wait_gather(k, b):
    pltpu.make_async_copy(x_hbm.at[src_all.at[pl.ds(k * C, C)]], rows[b],
                          gsem[b]).wait()

  def wait_scatter(b):
    pltpu.make_async_copy(rows[b], acc_sh.at[dstc[b]], ssem[b]).wait()

  def seg_body(g, carry):
    sbase = base + g * TBUF
    pltpu.sync_copy(src_hbm.at[pl.ds(sbase, TBUF)], src_all)
    pltpu.sync_copy(dst_hbm.at[pl.ds(sbase, TBUF)], dst_all)
    pltpu.sync_copy(w_hbm.at[pl.ds(sbase, TBUF)], w_all)
    start_gather(0, 0)
    start_gather(1, 1)

    def iter_body(t, carry2):
      for b in range(NSLOT):
        k = NSLOT * t + b
        wait_gather(k, b)
        scale_chunk(rows[b], k)
        stage_dst(b, k)
        pltpu.async_copy(rows[b], acc_sh.at[dstc[b]], ssem[b], add=True)
        b2 = (b + 2) % NSLOT
        if b < 2:
          @pl.when(t > 0)
          def _():
            wait_scatter(b2)
          start_gather(k + 2, b2)
        else:
          wait_scatter(b2)

          @pl.when(t < TSEG // NSLOT - 1)
          def _():
            start_gather(k + 2, b2)

      return carry2

    lax.fori_loop(0, TSEG // NSLOT, iter_body, 0)
    wait_scatter(2)
    wait_scatter(3)
    return carry

  lax.fori_loop(0, NSEG, seg_body, 0)

  @pl.when(wid < REM)
  def _():
    roff = base + FULL_T * C
    pltpu.sync_copy(src_hbm.at[pl.ds(roff, C)], src_all.at[pl.ds(0, C)])
    pltpu.sync_copy(dst_hbm.at[pl.ds(roff, C)], dst_all.at[pl.ds(0, C)])
    pltpu.sync_copy(w_hbm.at[pl.ds(roff, C)], w_all.at[pl.ds(0, C)])
    pltpu.async_copy(x_hbm.at[src_all.at[pl.ds(0, C)]], rows0, gsem0).wait()
    scale_chunk(rows0, 0)
    stage_dst(0, 0)
    pltpu.sync_copy(rows0, acc_sh.at[dstc0], add=True)

  plsc.subcore_barrier()
  rbase = s * ROWS_PER_TILE
  pltpu.sync_copy(acc_sh.at[pl.ds(rbase, ROWS_PER_TILE)],
                  part_hbm.at[c, pl.ds(rbase, ROWS_PER_TILE)])


_sc_agg = functools.partial(
    pl.kernel,
    out_type=jax.ShapeDtypeStruct((2, NPAD, D), jnp.float32),
    mesh=plsc.VectorSubcoreMesh(core_axis_name="c", subcore_axis_name="s"),
    scratch_types=(
        [pltpu.VMEM((TBUF,), jnp.int32),
         pltpu.VMEM((TBUF,), jnp.int32),
         pltpu.VMEM((TBUF,), jnp.float32)]
        + [pltpu.VMEM((C, D), jnp.float32) for _ in range(NSLOT)]
        + [pltpu.VMEM((C,), jnp.int32) for _ in range(NSLOT)]
        + [pltpu.VMEM_SHARED((NPAD, D), jnp.float32)]
        + [pltpu.SemaphoreType.DMA for _ in range(2 * NSLOT)]
    ),
)(_sc_body)


BM = 1000


def _mm_body(p_ref, w_ref, o_ref):
  acc = p_ref[0] + p_ref[1]
  o_ref[...] = jnp.dot(acc, w_ref[...], preferred_element_type=jnp.float32)


def _combine_matmul(part, W):
  return pl.pallas_call(
      _mm_body,
      grid=(N // BM,),
      in_specs=[
          pl.BlockSpec((2, BM, D), lambda i: (0, i, 0)),
          pl.BlockSpec((D, D), lambda i: (0, 0)),
      ],
      out_specs=pl.BlockSpec((BM, D), lambda i: (i, 0)),
      out_shape=jax.ShapeDtypeStruct((N, D), jnp.float32),
  )(part, W)


@jax.jit
def kernel(x, edge_index, edge_weight, W):
  dst = edge_index[0].astype(jnp.int32)
  src = edge_index[1].astype(jnp.int32)
  part = _sc_agg(x, src, dst, edge_weight)
  return _combine_matmul(part, W)
